# baseline (speedup 1.0000x reference)
import jax
import jax.numpy as jnp
from jax.experimental import pallas as pl
from jax.experimental.pallas import tpu as pltpu

_VMEM_LIMIT_BYTES = 56 << 20


def _mean_rows_kernel(x_ref, o_ref, *, inv_count):
    x = x_ref[...]
    part = jnp.sum(x.reshape(-1, 8, x.shape[-1]), axis=0)
    total = jnp.sum(part, axis=0, keepdims=True)
    o_ref[0] = (total * inv_count).astype(o_ref.dtype)


def kernel(tokens, outputs):
    del tokens
    B, S1, S2, D = outputs.shape
    R = S1 * S2
    x = outputs.reshape(B, R, D)

    out = pl.pallas_call(
        lambda x_ref, o_ref: _mean_rows_kernel(x_ref, o_ref, inv_count=1.0 / R),
        out_shape=jax.ShapeDtypeStruct((B, 1, D), outputs.dtype),
        grid_spec=pltpu.PrefetchScalarGridSpec(
            num_scalar_prefetch=0,
            grid=(B,),
            in_specs=[pl.BlockSpec((pl.Squeezed(), R, D), lambda b: (b, 0, 0))],
            out_specs=pl.BlockSpec((1, 1, D), lambda b: (b, 0, 0)),
        ),
        compiler_params=pltpu.CompilerParams(
            dimension_semantics=("parallel",),
            vmem_limit_bytes=_VMEM_LIMIT_BYTES,
        ),
    )(x)
    return out[:, 0, :]
